# baseline (speedup 1.0000x reference)
import functools

import jax
import jax.numpy as jnp
from jax.experimental import pallas as pl
from jax.experimental.pallas import tpu as pltpu


def _ceil_to(x: int, m: int) -> int:
    return ((x + m - 1) // m) * m


def _partial_table_kernel(emb_ref, w1_ref, b1_ref, w2_ref, part_ref):
    h = jnp.tanh(
        jnp.dot(emb_ref[...], w1_ref[...], preferred_element_type=jnp.float32)
        + b1_ref[...]
    )
    part_ref[0] = jnp.dot(h, w2_ref[...], preferred_element_type=jnp.float32)


_NBUF = 4


def _finalize_gather_kernel(idx_ref, part_ref, b2_ref, out_ref, t_ref,
                            ob0, ob1, ob2, ob3, sem_ref, *, n_half):
    i = pl.program_id(1)

    @pl.when(i == 0)
    def _build_table():
        t_ref[...] = (
            part_ref[0] + part_ref[1] + b2_ref[...]
        ).astype(t_ref.dtype)

    idx = idx_ref[...]
    tile_n = idx.shape[0]
    p_pad = t_ref.shape[0]
    col = jax.lax.broadcasted_iota(jnp.int32, (tile_n, p_pad), 1)
    onehot = (col == idx).astype(jnp.bfloat16)
    res = jnp.dot(onehot, t_ref[...], preferred_element_type=jnp.float32)

    row0 = (pl.program_id(0) * n_half + i) * tile_n
    row0 = pl.multiple_of(row0, 8)
    obs = (ob0, ob1, ob2, ob3)
    for k in range(_NBUF):
        @pl.when(i % _NBUF == k)
        def _emit(k=k):
            ob = obs[k]
            dst = out_ref.at[pl.ds(row0, tile_n), :]

            @pl.when(i >= _NBUF)
            def _drain_old():
                pltpu.make_async_copy(ob, dst, sem_ref.at[k]).wait()

            ob[...] = res
            pltpu.make_async_copy(ob, dst, sem_ref.at[k]).start()

    @pl.when(i == n_half - 1)
    def _drain_all():
        dst = out_ref.at[pl.ds(row0, tile_n), :]
        for k in range(min(_NBUF, n_half)):
            pltpu.make_async_copy(obs[k], dst, sem_ref.at[k]).wait()


@functools.partial(jax.jit, static_argnames=("tile_n",))
def _prefix_encoder(prefix, embedding, w1, b1, w2, b2, *, tile_n=1024):
    B, L = prefix.shape
    P, H = embedding.shape
    N = B * L

    f32 = jnp.float32
    bf16 = jnp.bfloat16

    h_pad = _ceil_to(H, 256)
    p_pad = _ceil_to(P, 8)
    hc = h_pad // 2
    tile_n = min(tile_n, _ceil_to(N, 8))
    n_half = pl.cdiv(pl.cdiv(N, tile_n), 2)
    n_pad = 2 * n_half * tile_n

    emb_p = jnp.pad(embedding.astype(f32), ((0, p_pad - P), (0, h_pad - H)))
    w1_p = jnp.pad(w1.astype(f32), ((0, h_pad - H), (0, h_pad - H)))
    w2_p = jnp.pad(w2.astype(f32), ((0, h_pad - H), (0, h_pad - H)))
    b1_p = jnp.pad(b1.astype(f32), (0, h_pad - H)).reshape(1, h_pad)
    b2_p = jnp.pad(b2.astype(f32), (0, h_pad - H)).reshape(1, h_pad)

    partials = pl.pallas_call(
        _partial_table_kernel,
        out_shape=jax.ShapeDtypeStruct((2, p_pad, h_pad), f32),
        grid=(2,),
        in_specs=[
            pl.BlockSpec((p_pad, h_pad), lambda j: (0, 0)),
            pl.BlockSpec((h_pad, hc), lambda j: (0, j)),
            pl.BlockSpec((1, hc), lambda j: (0, j)),
            pl.BlockSpec((hc, h_pad), lambda j: (j, 0)),
        ],
        out_specs=pl.BlockSpec((1, p_pad, h_pad), lambda j: (j, 0, 0)),
        compiler_params=pltpu.CompilerParams(
            dimension_semantics=("parallel",),
            vmem_limit_bytes=48 * 1024 * 1024),
        cost_estimate=pl.CostEstimate(
            flops=4 * p_pad * h_pad * h_pad,
            transcendentals=p_pad * h_pad,
            bytes_accessed=4 * (p_pad * h_pad + 2 * h_pad * h_pad + h_pad
                                + 2 * p_pad * h_pad)),
    )(emb_p, w1_p, b1_p, w2_p)

    idx = jnp.pad(prefix.reshape(N).astype(jnp.int32), (0, n_pad - N))
    idx = idx.reshape(n_pad, 1)

    out = pl.pallas_call(
        functools.partial(_finalize_gather_kernel, n_half=n_half),
        out_shape=jax.ShapeDtypeStruct((n_pad, h_pad), f32),
        grid=(2, n_half),
        in_specs=[
            pl.BlockSpec((tile_n, 1),
                         lambda j, i, nh=n_half: (j * nh + i, 0)),
            pl.BlockSpec((2, p_pad, h_pad), lambda j, i: (0, 0, 0)),
            pl.BlockSpec((1, h_pad), lambda j, i: (0, 0)),
        ],
        out_specs=pl.BlockSpec(memory_space=pl.ANY),
        scratch_shapes=[
            pltpu.VMEM((p_pad, h_pad), bf16),
        ] + [pltpu.VMEM((tile_n, h_pad), f32) for _ in range(_NBUF)] + [
            pltpu.SemaphoreType.DMA((_NBUF,)),
        ],
        compiler_params=pltpu.CompilerParams(
            dimension_semantics=("parallel", "arbitrary"),
            vmem_limit_bytes=48 * 1024 * 1024),
        cost_estimate=pl.CostEstimate(
            flops=2 * n_pad * p_pad * h_pad,
            transcendentals=0,
            bytes_accessed=4 * n_pad * (1 + h_pad) + 8 * p_pad * h_pad),
    )(idx, partials, b2_p)

    return out[:N, :H].reshape(B, L, H)


def kernel(prefix, embedding, w1, b1, w2, b2):
    return _prefix_encoder(prefix, embedding, w1, b1, w2, b2)


# speedup vs baseline: 1.0357x; 1.0357x over previous
import functools

import jax
import jax.numpy as jnp
from jax.experimental import pallas as pl
from jax.experimental.pallas import tpu as pltpu


def _ceil_to(x: int, m: int) -> int:
    return ((x + m - 1) // m) * m


def _hidden_kernel(emb_ref, w1_ref, b1_ref, h_ref):
    h_ref[...] = jnp.tanh(
        jnp.dot(emb_ref[...], w1_ref[...], preferred_element_type=jnp.float32)
        + b1_ref[...]
    ).astype(h_ref.dtype)


def _table_kernel(h_ref, w2_ref, b2_ref, t_ref):
    acc = jnp.dot(
        h_ref[...].astype(jnp.float32), w2_ref[...],
        preferred_element_type=jnp.float32,
    )
    t_ref[...] = (acc + b2_ref[...]).astype(t_ref.dtype)


def _onehot_gather_kernel(idx_ref, t_ref, out_ref):
    idx = idx_ref[...]
    tile_n = idx.shape[0]
    p_pad = t_ref.shape[0]
    col = jax.lax.broadcasted_iota(jnp.int32, (tile_n, p_pad), 1)
    onehot = (col == idx).astype(jnp.bfloat16)
    out_ref[...] = jnp.dot(
        onehot, t_ref[...], preferred_element_type=jnp.float32
    )


@functools.partial(jax.jit, static_argnames=("tile_n",))
def _prefix_encoder(prefix, embedding, w1, b1, w2, b2, *, tile_n=2048):
    B, L = prefix.shape
    P, H = embedding.shape
    N = B * L

    f32 = jnp.float32
    bf16 = jnp.bfloat16

    h_pad = _ceil_to(H, 256)
    p_pad = _ceil_to(P, 8)
    hc = h_pad // 2
    tile_n = min(tile_n, _ceil_to(N, 8))
    n_tiles = pl.cdiv(N, tile_n)
    n_pad = n_tiles * tile_n

    emb_p = jnp.pad(embedding.astype(f32), ((0, p_pad - P), (0, h_pad - H)))
    w1_p = jnp.pad(w1.astype(f32), ((0, h_pad - H), (0, h_pad - H)))
    w2_p = jnp.pad(w2.astype(f32), ((0, h_pad - H), (0, h_pad - H)))
    b1_p = jnp.pad(b1.astype(f32), (0, h_pad - H)).reshape(1, h_pad)
    b2_p = jnp.pad(b2.astype(f32), (0, h_pad - H)).reshape(1, h_pad)

    hidden = pl.pallas_call(
        _hidden_kernel,
        out_shape=jax.ShapeDtypeStruct((p_pad, h_pad), bf16),
        grid=(2,),
        in_specs=[
            pl.BlockSpec((p_pad, h_pad), lambda j: (0, 0)),
            pl.BlockSpec((h_pad, hc), lambda j: (0, j)),
            pl.BlockSpec((1, hc), lambda j: (0, j)),
        ],
        out_specs=pl.BlockSpec((p_pad, hc), lambda j: (0, j)),
        compiler_params=pltpu.CompilerParams(
            dimension_semantics=("parallel",),
            vmem_limit_bytes=48 * 1024 * 1024),
        cost_estimate=pl.CostEstimate(
            flops=2 * p_pad * h_pad * h_pad,
            transcendentals=p_pad * h_pad,
            bytes_accessed=4 * (p_pad * h_pad + h_pad * h_pad + h_pad)
            + 2 * p_pad * h_pad),
    )(emb_p, w1_p, b1_p)

    table = pl.pallas_call(
        _table_kernel,
        out_shape=jax.ShapeDtypeStruct((p_pad, h_pad), bf16),
        grid=(2,),
        in_specs=[
            pl.BlockSpec((p_pad, h_pad), lambda j: (0, 0)),
            pl.BlockSpec((h_pad, hc), lambda j: (0, j)),
            pl.BlockSpec((1, hc), lambda j: (0, j)),
        ],
        out_specs=pl.BlockSpec((p_pad, hc), lambda j: (0, j)),
        compiler_params=pltpu.CompilerParams(
            dimension_semantics=("parallel",),
            vmem_limit_bytes=48 * 1024 * 1024),
        cost_estimate=pl.CostEstimate(
            flops=2 * p_pad * h_pad * h_pad,
            transcendentals=0,
            bytes_accessed=4 * (h_pad * h_pad + h_pad)
            + 2 * (2 * p_pad * h_pad)),
    )(hidden, w2_p, b2_p)

    idx = jnp.pad(prefix.reshape(N).astype(jnp.int32), (0, n_pad - N))
    idx = idx.reshape(n_pad, 1)

    out = pl.pallas_call(
        _onehot_gather_kernel,
        out_shape=jax.ShapeDtypeStruct((n_pad, h_pad), f32),
        grid=(n_tiles,),
        in_specs=[
            pl.BlockSpec((tile_n, 1), lambda i: (i, 0)),
            pl.BlockSpec((p_pad, h_pad), lambda i: (0, 0)),
        ],
        out_specs=pl.BlockSpec((tile_n, h_pad), lambda i: (i, 0)),
        compiler_params=pltpu.CompilerParams(
            dimension_semantics=("parallel",),
            vmem_limit_bytes=48 * 1024 * 1024),
        cost_estimate=pl.CostEstimate(
            flops=2 * n_pad * p_pad * h_pad,
            transcendentals=0,
            bytes_accessed=4 * n_pad * (1 + h_pad) + 2 * p_pad * h_pad),
    )(idx, table)

    return out[:N, :H].reshape(B, L, H)


def kernel(prefix, embedding, w1, b1, w2, b2):
    return _prefix_encoder(prefix, embedding, w1, b1, w2, b2)
